# baseline (speedup 1.0000x reference)
import jax
import jax.numpy as jnp
import numpy as np
from jax import lax
from jax.experimental import pallas as pl
from jax.experimental.pallas import tpu as pltpu
from jax.experimental.pallas import tpu_sc as plsc

N = 10000
G = 64
EPS = 1e-5
NP = 10240
LN = 128
TILES = 32
EPT = 5120
NB = EPT // LN
E_PAD = TILES * EPT
ROWS_PT = NP // 16
CPT = G * NP // 16

_mesh = plsc.VectorSubcoreMesh(core_axis_name="c", subcore_axis_name="s")


def _sc_hist_body(srcr, dstr, eor, batchr, z1dr,
                  deg_o, c_o,
                  src_v, dst_v, eo_v, cidx_v, gk_v, z1d_v,
                  hsem0, hsem1, cS, degS):
    c = lax.axis_index("c")
    s = lax.axis_index("s")
    t = c * 16 + s

    pltpu.sync_copy(srcr.at[pl.ds(t * NB, NB)], src_v)
    pltpu.sync_copy(dstr.at[pl.ds(t * NB, NB)], dst_v)
    pltpu.sync_copy(eor.at[pl.ds(t * NB, NB)], eo_v)
    pltpu.sync_copy(z1dr, z1d_v)

    def gk_fire(j, _):
        pltpu.async_copy(batchr.at[dst_v.at[j]], gk_v.at[j], hsem0)
        return 0

    lax.fori_loop(0, NB, gk_fire, 0)

    def gk_drain(j, _):
        pltpu.make_async_copy(batchr.at[dst_v.at[j]], gk_v.at[j],
                              hsem0).wait()
        return 0

    lax.fori_loop(0, NB, gk_drain, 0)

    def cidx_body(i, _):
        j = i // 8
        k = (i % 8) * 16
        cidx_v[j, pl.ds(k, 16)] = (gk_v[j, pl.ds(k, 16)] * NP
                                   + src_v[j, pl.ds(k, 16)])
        return 0

    lax.fori_loop(0, NB * 8, cidx_body, 0)

    pltpu.sync_copy(z1d_v.at[pl.ds(0, ROWS_PT)],
                    degS.at[pl.ds(s * ROWS_PT, ROWS_PT)])

    def zc(i, _):
        pltpu.sync_copy(z1d_v, cS.at[pl.ds(s * CPT + i * 4096, 4096)])
        return 0

    lax.fori_loop(0, CPT // 4096, zc, 0)
    plsc.subcore_barrier()

    def dc_fire(j, _):
        pltpu.async_copy(eo_v.at[j], degS.at[dst_v.at[j]], hsem0, add=True)
        pltpu.async_copy(eo_v.at[j], cS.at[cidx_v.at[j]], hsem1, add=True)
        return 0

    lax.fori_loop(0, NB, dc_fire, 0)

    def dc_drain(j, _):
        pltpu.make_async_copy(eo_v.at[j], degS.at[dst_v.at[j]],
                              hsem0).wait()
        pltpu.make_async_copy(eo_v.at[j], cS.at[cidx_v.at[j]],
                              hsem1).wait()
        return 0

    lax.fori_loop(0, NB, dc_drain, 0)
    plsc.subcore_barrier()
    pltpu.sync_copy(degS.at[pl.ds(s * ROWS_PT, ROWS_PT)],
                    deg_o.at[c, pl.ds(s * ROWS_PT, ROWS_PT)])

    def wc(i, _):
        off = s * CPT + i * 4096
        pltpu.sync_copy(cS.at[pl.ds(off, 4096)], c_o.at[c, pl.ds(off, 4096)])
        return 0

    lax.fori_loop(0, CPT // 4096, wc, 0)


def _sc_hist(src_p, dst_p, eones, batchp, z1d):
    f = pl.kernel(
        _sc_hist_body,
        out_type=(
            jax.ShapeDtypeStruct((2, NP), jnp.float32),
            jax.ShapeDtypeStruct((2, G * NP), jnp.float32),
        ),
        mesh=_mesh,
        scratch_types=[
            pltpu.VMEM((NB, LN), jnp.int32),
            pltpu.VMEM((NB, LN), jnp.int32),
            pltpu.VMEM((NB, LN), jnp.float32),
            pltpu.VMEM((NB, LN), jnp.int32),
            pltpu.VMEM((NB, LN), jnp.int32),
            pltpu.VMEM((4096,), jnp.float32),
            pltpu.SemaphoreType.DMA,
            pltpu.SemaphoreType.DMA,
            pltpu.VMEM_SHARED((G * NP,), jnp.float32),
            pltpu.VMEM_SHARED((NP,), jnp.float32),
        ],
    )
    return f(src_p, dst_p, eones, batchp, z1d)


def _make_sc_agg(nch):
    def body(table, srcr, dstr, z2dr, agg_o,
             src_v, dst_v, gidx_v, rows0, rows1, gsem0, gsem1, accS):
        c = lax.axis_index("c")
        s = lax.axis_index("s")
        t = c * 16 + s

        pltpu.sync_copy(srcr.at[pl.ds(t * NB, NB)], src_v)
        pltpu.sync_copy(dstr.at[pl.ds(t * NB, NB)], dst_v)

        for ch in range(nch):
            def gidx_body(i, _):
                j = i // 8
                k = (i % 8) * 16
                gidx_v[j, pl.ds(k, 16)] = src_v[j, pl.ds(k, 16)] * nch + ch
                return 0

            lax.fori_loop(0, NB * 8, gidx_body, 0)

            def zacc(i, _):
                pltpu.sync_copy(z2dr,
                                accS.at[pl.ds(s * ROWS_PT + i * 64, 64), :])
                return 0

            lax.fori_loop(0, ROWS_PT // 64, zacc, 0)
            plsc.subcore_barrier()

            pltpu.async_copy(table.at[gidx_v.at[0]], rows0, gsem0)

            def grp(g, _):
                j0 = 2 * g
                j1 = 2 * g + 1
                pltpu.async_copy(table.at[gidx_v.at[j1]], rows1, gsem1)
                pltpu.make_async_copy(table.at[gidx_v.at[j0]], rows0,
                                      gsem0).wait()
                pltpu.sync_copy(rows0, accS.at[dst_v.at[j0]], add=True)

                @pl.when(j1 + 1 < NB)
                def _():
                    pltpu.async_copy(table.at[gidx_v.at[j1 + 1]], rows0,
                                     gsem0)

                pltpu.make_async_copy(table.at[gidx_v.at[j1]], rows1,
                                      gsem1).wait()
                pltpu.sync_copy(rows1, accS.at[dst_v.at[j1]], add=True)
                return 0

            lax.fori_loop(0, NB // 2, grp, 0)
            plsc.subcore_barrier()
            pltpu.sync_copy(accS.at[pl.ds(s * ROWS_PT, ROWS_PT), :],
                            agg_o.at[c, ch, pl.ds(s * ROWS_PT, ROWS_PT), :])

    def run(table, src_p, dst_p, z2d):
        f = pl.kernel(
            body,
            out_type=jax.ShapeDtypeStruct((2, nch, NP, LN), jnp.float32),
            mesh=_mesh,
            scratch_types=[
                pltpu.VMEM((NB, LN), jnp.int32),
                pltpu.VMEM((NB, LN), jnp.int32),
                pltpu.VMEM((NB, LN), jnp.int32),
                pltpu.VMEM((LN, LN), jnp.float32),
                pltpu.VMEM((LN, LN), jnp.float32),
                pltpu.SemaphoreType.DMA,
                pltpu.SemaphoreType.DMA,
                pltpu.VMEM_SHARED((NP, LN), jnp.float32),
            ],
        )
        return f(table, src_p, dst_p, z2d)

    return run


_sc_agg2 = _make_sc_agg(2)
_sc_agg4 = _make_sc_agg(4)


def _tc1_body(agg_ref, xp_ref, deg_ref, L0_ref, R0_ref, cbd_ref, cbc_ref,
              s0_ref, b0_ref, h1_ref):
    i = pl.program_id(0)
    z = jnp.dot(agg_ref[0] + agg_ref[2], L0_ref[0],
                preferred_element_type=jnp.float32)
    z += jnp.dot(agg_ref[1] + agg_ref[3], L0_ref[1],
                 preferred_element_type=jnp.float32)
    z += jnp.dot(xp_ref[...], R0_ref[...], preferred_element_type=jnp.float32)
    deg = deg_ref[0] + deg_ref[1]
    z += deg[:, None] * cbd_ref[...][None, :] + cbc_ref[...][None, :]
    h = jax.nn.relu(z * s0_ref[...][None, :] + b0_ref[...][None, :])
    rid = i * 256 + lax.broadcasted_iota(jnp.int32, (256, 1), 0)
    h1_ref[...] = jnp.where(rid < N, h, 0.0)


def _tc1(aggx_p, xp, deg_p, L0, R0, cbd, cbc, s0, b0):
    return pl.pallas_call(
        _tc1_body,
        grid=(NP // 256,),
        in_specs=[
            pl.BlockSpec((4, 256, LN), lambda i: (0, i, 0)),
            pl.BlockSpec((256, 256), lambda i: (i, 0)),
            pl.BlockSpec((2, 256), lambda i: (0, i)),
            pl.BlockSpec((2, LN, 512), lambda i: (0, 0, 0)),
            pl.BlockSpec((256, 512), lambda i: (0, 0)),
            pl.BlockSpec((512,), lambda i: (0,)),
            pl.BlockSpec((512,), lambda i: (0,)),
            pl.BlockSpec((512,), lambda i: (0,)),
            pl.BlockSpec((512,), lambda i: (0,)),
        ],
        out_specs=pl.BlockSpec((256, 512), lambda i: (i, 0)),
        out_shape=jax.ShapeDtypeStruct((NP, 512), jnp.float32),
    )(aggx_p, xp, deg_p, L0, R0, cbd, cbc, s0, b0)


def _tc2_body(agg_ref, h1_ref, deg_ref, L1_ref, R1_ref, l1b_ref,
              s1_ref, b1_ref, h2_ref):
    i = pl.program_id(0)
    deg = deg_ref[0] + deg_ref[1]
    idg = 1.0 / jnp.maximum(deg, 1.0)
    z = jnp.dot(h1_ref[...], R1_ref[...], preferred_element_type=jnp.float32)
    for ch in range(4):
        a = (agg_ref[ch] + agg_ref[ch + 4]) * idg[:, None]
        z += jnp.dot(a, L1_ref[ch], preferred_element_type=jnp.float32)
    z += l1b_ref[...][None, :]
    h = jax.nn.relu(z * s1_ref[...][None, :] + b1_ref[...][None, :])
    rid = i * 256 + lax.broadcasted_iota(jnp.int32, (256, 1), 0)
    h2_ref[...] = jnp.where(rid < N, h, 0.0)


def _tc2(agg1_p, h1, deg_p, L1, R1, l1b, s1, b1):
    return pl.pallas_call(
        _tc2_body,
        grid=(NP // 256,),
        in_specs=[
            pl.BlockSpec((8, 256, LN), lambda i: (0, i, 0)),
            pl.BlockSpec((256, 512), lambda i: (i, 0)),
            pl.BlockSpec((2, 256), lambda i: (0, i)),
            pl.BlockSpec((4, LN, 512), lambda i: (0, 0, 0)),
            pl.BlockSpec((512, 512), lambda i: (0, 0)),
            pl.BlockSpec((512,), lambda i: (0,)),
            pl.BlockSpec((512,), lambda i: (0,)),
            pl.BlockSpec((512,), lambda i: (0,)),
        ],
        out_specs=pl.BlockSpec((256, 512), lambda i: (i, 0)),
        out_shape=jax.ShapeDtypeStruct((NP, 512), jnp.float32),
    )(agg1_p, h1, deg_p, L1, R1, l1b, s1, b1)


def _tc3_body(h2_ref, batch_ref, c_ref, wl_ref, wr_ref, bb_ref, cb_ref,
              out_ref, acc1, acc2, accn):
    i = pl.program_id(0)

    @pl.when(i == 0)
    def _():
        acc1[...] = jnp.zeros_like(acc1)
        acc2[...] = jnp.zeros_like(acc2)
        accn[...] = jnp.zeros_like(accn)

    bb = batch_ref[...]
    gi = lax.broadcasted_iota(jnp.int32, (512, G), 1)
    oh = (bb[:, None] == gi).astype(jnp.float32)
    h2 = h2_ref[...]
    acc2[...] += lax.dot_general(oh, h2, (((0,), (0,)), ((), ())),
                                 preferred_element_type=jnp.float32)
    cb = c_ref[0] + c_ref[1]
    acc1[...] += lax.dot_general(cb, h2, (((1,), (0,)), ((), ())),
                                 preferred_element_type=jnp.float32)
    accn[...] += jnp.sum(oh, axis=0, keepdims=True)

    @pl.when(i == NP // 512 - 1)
    def _():
        cnt = accn[0]
        icnt = 1.0 / jnp.maximum(cnt, 1.0)
        nz = (cnt > 0).astype(jnp.float32)
        p1 = acc1[...] * icnt[:, None]
        p2 = acc2[...] * icnt[:, None]
        o = jnp.dot(p1, wl_ref[...], preferred_element_type=jnp.float32)
        o += jnp.dot(p2, wr_ref[...], preferred_element_type=jnp.float32)
        o += nz[:, None] * bb_ref[...][None, :] + cb_ref[...][None, :]
        out_ref[...] = o


def _tc3(h2, batchp, C_p, WlT, WrT, bb2, clsb):
    return pl.pallas_call(
        _tc3_body,
        grid=(NP // 512,),
        in_specs=[
            pl.BlockSpec((512, 512), lambda i: (i, 0)),
            pl.BlockSpec((512,), lambda i: (i,)),
            pl.BlockSpec((2, G, 512), lambda i: (0, 0, i)),
            pl.BlockSpec((512, 16), lambda i: (0, 0)),
            pl.BlockSpec((512, 16), lambda i: (0, 0)),
            pl.BlockSpec((16,), lambda i: (0,)),
            pl.BlockSpec((16,), lambda i: (0,)),
        ],
        out_specs=pl.BlockSpec((G, 16), lambda i: (0, 0)),
        out_shape=jax.ShapeDtypeStruct((G, 16), jnp.float32),
        scratch_shapes=[
            pltpu.VMEM((G, 512), jnp.float32),
            pltpu.VMEM((G, 512), jnp.float32),
            pltpu.VMEM((1, G), jnp.float32),
        ],
    )(h2, batchp, C_p, WlT, WrT, bb2, clsb)


def kernel(x, edge_index, edge_attr, batch, w_emb, b_emb,
           conv0_l_w, conv0_l_b, conv0_r_w,
           conv1_l_w, conv1_l_b, conv1_r_w,
           conv2_l_w, conv2_l_b, conv2_r_w,
           bn0_g, bn0_b, bn1_g, bn1_b, cls_w, cls_b):
    E = edge_index.shape[1]
    dst0, src0 = jax.lax.sort([edge_index[1].astype(jnp.int32),
                               edge_index[0].astype(jnp.int32)],
                              num_keys=1)
    src = src0
    dst = dst0
    npad = E_PAD - E
    src_p = jnp.concatenate(
        [src, N + (jnp.arange(npad, dtype=jnp.int32) % 16)]
    ).reshape(TILES * NB, LN)
    dst_p = jnp.concatenate(
        [dst, jnp.zeros((npad,), jnp.int32)]).reshape(TILES * NB, LN)
    eones = jnp.concatenate(
        [jnp.ones((E,), jnp.float32), jnp.zeros((npad,), jnp.float32)]
    ).reshape(TILES * NB, LN)
    xp = jnp.pad(x, ((0, NP - N), (0, 0)))
    xp2 = xp.reshape(2 * NP, LN)
    batchp = jnp.pad(batch.astype(jnp.int32), (0, NP - N), constant_values=G)
    z2d = jnp.zeros((64, LN), jnp.float32)
    z1d = jnp.zeros((4096,), jnp.float32)

    inv = 1.0 / np.sqrt(1.0 + EPS)
    L0 = jnp.concatenate([conv0_l_w[:, :10] @ w_emb, conv0_l_w[:, 10:]], axis=1)
    R0 = jnp.concatenate([conv0_r_w[:, :10] @ w_emb, conv0_r_w[:, 10:]], axis=1)
    cbd = conv0_l_w[:, :10] @ b_emb
    cbc = conv0_r_w[:, :10] @ b_emb + conv0_l_b
    s0 = bn0_g * inv
    s1 = bn1_g * inv
    WlT = (cls_w @ conv2_l_w).T
    WrT = (cls_w @ conv2_r_w).T
    bb2 = cls_w @ conv2_l_b

    deg_p, C_p = _sc_hist(src_p, dst_p, eones, batchp, z1d)
    aggx_p = _sc_agg2(xp2, src_p, dst_p, z2d)
    h1 = _tc1(aggx_p.reshape(4, NP, LN), xp, deg_p,
              L0.T.reshape(2, LN, 512), R0.T, cbd, cbc, s0, bn0_b)
    agg1_p = _sc_agg4(h1.reshape(4 * NP, LN), src_p, dst_p, z2d)
    h2 = _tc2(agg1_p.reshape(8, NP, LN), h1, deg_p,
              conv1_l_w.T.reshape(4, LN, 512), conv1_r_w.T, conv1_l_b,
              s1, bn1_b)
    out = _tc3(h2, batchp, C_p.reshape(2, G, NP), WlT, WrT, bb2, cls_b)
    return out


# speedup vs baseline: 1.3620x; 1.3620x over previous
import jax
import jax.numpy as jnp
import numpy as np
from jax import lax
from jax.experimental import pallas as pl
from jax.experimental.pallas import tpu as pltpu
from jax.experimental.pallas import tpu_sc as plsc

N = 10000
G = 64
EPS = 1e-5
NP = 10240
LN = 128
TILES = 32
EPT = 5120
NB = EPT // LN
E_PAD = TILES * EPT
ROWS_PT = NP // 16
CPT = G * NP // 16

_mesh = plsc.VectorSubcoreMesh(core_axis_name="c", subcore_axis_name="s")


def _sc_hist_body(srcr, dstr, eor, batchr, z1dr,
                  deg_o, c_o,
                  src_v, dst_v, eo_v, cidx_v, gk_v, z1d_v,
                  hsem0, hsem1, cS, degS):
    c = lax.axis_index("c")
    s = lax.axis_index("s")
    t = c * 16 + s

    pltpu.sync_copy(srcr.at[pl.ds(t * NB, NB)], src_v)
    pltpu.sync_copy(dstr.at[pl.ds(t * NB, NB)], dst_v)
    pltpu.sync_copy(eor.at[pl.ds(t * NB, NB)], eo_v)
    pltpu.sync_copy(z1dr, z1d_v)

    def gk_fire(j, _):
        pltpu.async_copy(batchr.at[dst_v.at[j]], gk_v.at[j], hsem0)
        return 0

    lax.fori_loop(0, NB, gk_fire, 0)

    def gk_drain(j, _):
        pltpu.make_async_copy(batchr.at[dst_v.at[j]], gk_v.at[j],
                              hsem0).wait()
        return 0

    lax.fori_loop(0, NB, gk_drain, 0)

    def cidx_body(i, _):
        j = i // 8
        k = (i % 8) * 16
        cidx_v[j, pl.ds(k, 16)] = (gk_v[j, pl.ds(k, 16)] * NP
                                   + src_v[j, pl.ds(k, 16)])
        return 0

    lax.fori_loop(0, NB * 8, cidx_body, 0)

    pltpu.sync_copy(z1d_v.at[pl.ds(0, ROWS_PT)],
                    degS.at[pl.ds(s * ROWS_PT, ROWS_PT)])

    def zc(i, _):
        pltpu.sync_copy(z1d_v, cS.at[pl.ds(s * CPT + i * 4096, 4096)])
        return 0

    lax.fori_loop(0, CPT // 4096, zc, 0)
    plsc.subcore_barrier()

    def dc_fire(j, _):
        pltpu.async_copy(eo_v.at[j], degS.at[dst_v.at[j]], hsem0, add=True)
        pltpu.async_copy(eo_v.at[j], cS.at[cidx_v.at[j]], hsem1, add=True)
        return 0

    lax.fori_loop(0, NB, dc_fire, 0)

    def dc_drain(j, _):
        pltpu.make_async_copy(eo_v.at[j], degS.at[dst_v.at[j]],
                              hsem0).wait()
        pltpu.make_async_copy(eo_v.at[j], cS.at[cidx_v.at[j]],
                              hsem1).wait()
        return 0

    lax.fori_loop(0, NB, dc_drain, 0)
    plsc.subcore_barrier()
    pltpu.sync_copy(degS.at[pl.ds(s * ROWS_PT, ROWS_PT)],
                    deg_o.at[c, pl.ds(s * ROWS_PT, ROWS_PT)])

    def wc(i, _):
        off = s * CPT + i * 4096
        pltpu.sync_copy(cS.at[pl.ds(off, 4096)], c_o.at[c, pl.ds(off, 4096)])
        return 0

    lax.fori_loop(0, CPT // 4096, wc, 0)


def _sc_hist(src_p, dst_p, eones, batchp, z1d):
    f = pl.kernel(
        _sc_hist_body,
        out_type=(
            jax.ShapeDtypeStruct((2, NP), jnp.float32),
            jax.ShapeDtypeStruct((2, G * NP), jnp.float32),
        ),
        mesh=_mesh,
        scratch_types=[
            pltpu.VMEM((NB, LN), jnp.int32),
            pltpu.VMEM((NB, LN), jnp.int32),
            pltpu.VMEM((NB, LN), jnp.float32),
            pltpu.VMEM((NB, LN), jnp.int32),
            pltpu.VMEM((NB, LN), jnp.int32),
            pltpu.VMEM((4096,), jnp.float32),
            pltpu.SemaphoreType.DMA,
            pltpu.SemaphoreType.DMA,
            pltpu.VMEM_SHARED((G * NP,), jnp.float32),
            pltpu.VMEM_SHARED((NP,), jnp.float32),
        ],
    )
    return f(src_p, dst_p, eones, batchp, z1d)


def _make_sc_agg(nch, dtype=jnp.float32, W=LN):
    def body(table, srcr, dstr, z2dr, agg_o,
             src_v, dst_v, gidx_v, rows0, rows1, gsem0, gsem1, accS):
        c = lax.axis_index("c")
        s = lax.axis_index("s")
        t = c * 16 + s

        pltpu.sync_copy(srcr.at[pl.ds(t * NB, NB)], src_v)
        pltpu.sync_copy(dstr.at[pl.ds(t * NB, NB)], dst_v)

        for ch in range(nch):
            def gidx_body(i, _):
                j = i // 8
                k = (i % 8) * 16
                gidx_v[j, pl.ds(k, 16)] = src_v[j, pl.ds(k, 16)] * nch + ch
                return 0

            lax.fori_loop(0, NB * 8, gidx_body, 0)

            def zacc(i, _):
                pltpu.sync_copy(z2dr,
                                accS.at[pl.ds(s * ROWS_PT + i * 64, 64), :])
                return 0

            lax.fori_loop(0, ROWS_PT // 64, zacc, 0)
            plsc.subcore_barrier()

            pltpu.async_copy(table.at[gidx_v.at[0]], rows0, gsem0)

            def grp(g, _):
                j0 = 2 * g
                j1 = 2 * g + 1
                pltpu.async_copy(table.at[gidx_v.at[j1]], rows1, gsem1)
                pltpu.make_async_copy(table.at[gidx_v.at[j0]], rows0,
                                      gsem0).wait()
                pltpu.sync_copy(rows0, accS.at[dst_v.at[j0]], add=True)

                @pl.when(j1 + 1 < NB)
                def _():
                    pltpu.async_copy(table.at[gidx_v.at[j1 + 1]], rows0,
                                     gsem0)

                pltpu.make_async_copy(table.at[gidx_v.at[j1]], rows1,
                                      gsem1).wait()
                pltpu.sync_copy(rows1, accS.at[dst_v.at[j1]], add=True)
                return 0

            lax.fori_loop(0, NB // 2, grp, 0)
            plsc.subcore_barrier()
            pltpu.sync_copy(accS.at[pl.ds(s * ROWS_PT, ROWS_PT), :],
                            agg_o.at[c, ch, pl.ds(s * ROWS_PT, ROWS_PT), :])

    def run(table, src_p, dst_p, z2d):
        f = pl.kernel(
            body,
            out_type=jax.ShapeDtypeStruct((2, nch, NP, W), dtype),
            mesh=_mesh,
            scratch_types=[
                pltpu.VMEM((NB, LN), jnp.int32),
                pltpu.VMEM((NB, LN), jnp.int32),
                pltpu.VMEM((NB, LN), jnp.int32),
                pltpu.VMEM((LN, W), dtype),
                pltpu.VMEM((LN, W), dtype),
                pltpu.SemaphoreType.DMA,
                pltpu.SemaphoreType.DMA,
                pltpu.VMEM_SHARED((NP, W), dtype),
            ],
        )
        return f(table, src_p, dst_p, z2d)

    return run


_sc_agg2 = _make_sc_agg(2)
_sc_agg4 = _make_sc_agg(4)


def _tc1_body(agg_ref, xp_ref, deg_ref, L0_ref, R0_ref, cbd_ref, cbc_ref,
              s0_ref, b0_ref, h1_ref):
    i = pl.program_id(0)
    a0 = agg_ref[0] + agg_ref[2]
    a1 = agg_ref[1] + agg_ref[3]
    z = jnp.dot(a0, L0_ref[0], preferred_element_type=jnp.float32)
    z += jnp.dot(a1, L0_ref[1], preferred_element_type=jnp.float32)
    z += jnp.dot(xp_ref[...], R0_ref[...], preferred_element_type=jnp.float32)
    deg = deg_ref[0] + deg_ref[1]
    z += deg[:, None] * cbd_ref[...][None, :] + cbc_ref[...][None, :]
    h = jax.nn.relu(z * s0_ref[...][None, :] + b0_ref[...][None, :])
    rid = i * 256 + lax.broadcasted_iota(jnp.int32, (256, 1), 0)
    h1_ref[...] = jnp.where(rid < N, h, 0.0)


def _tc1(aggx_p, xp, deg_p, L0, R0, cbd, cbc, s0, b0):
    return pl.pallas_call(
        _tc1_body,
        grid=(NP // 256,),
        in_specs=[
            pl.BlockSpec((4, 256, LN), lambda i: (0, i, 0)),
            pl.BlockSpec((256, 256), lambda i: (i, 0)),
            pl.BlockSpec((2, 256), lambda i: (0, i)),
            pl.BlockSpec((2, LN, 512), lambda i: (0, 0, 0)),
            pl.BlockSpec((256, 512), lambda i: (0, 0)),
            pl.BlockSpec((512,), lambda i: (0,)),
            pl.BlockSpec((512,), lambda i: (0,)),
            pl.BlockSpec((512,), lambda i: (0,)),
            pl.BlockSpec((512,), lambda i: (0,)),
        ],
        out_specs=pl.BlockSpec((256, 512), lambda i: (i, 0)),
        out_shape=jax.ShapeDtypeStruct((NP, 512), jnp.float32),
    )(aggx_p, xp, deg_p, L0, R0, cbd, cbc, s0, b0)


def _tc23_body(agg_ref, h1_ref, deg_ref, L1_ref, R1_ref, l1b_ref,
               s1_ref, b1_ref, batch_ref, c_ref, wl_ref, wr_ref, bb_ref,
               cb_ref, out_ref, acc1, acc2, accn):
    i = pl.program_id(0)

    @pl.when(i == 0)
    def _():
        acc1[...] = jnp.zeros_like(acc1)
        acc2[...] = jnp.zeros_like(acc2)
        accn[...] = jnp.zeros_like(accn)

    deg = deg_ref[0] + deg_ref[1]
    idg = 1.0 / jnp.maximum(deg, 1.0)
    z = jnp.dot(h1_ref[...], R1_ref[...], preferred_element_type=jnp.float32)
    for ch in range(4):
        a = (agg_ref[ch] + agg_ref[ch + 4]) * idg[:, None]
        z += jnp.dot(a, L1_ref[ch], preferred_element_type=jnp.float32)
    z += l1b_ref[...][None, :]
    h = jax.nn.relu(z * s1_ref[...][None, :] + b1_ref[...][None, :])
    rid = i * 256 + lax.broadcasted_iota(jnp.int32, (256, 1), 0)
    h2 = jnp.where(rid < N, h, 0.0)

    bb = batch_ref[...]
    gi = lax.broadcasted_iota(jnp.int32, (256, G), 1)
    oh = (bb[:, None] == gi).astype(jnp.float32)
    acc2[...] += lax.dot_general(oh, h2, (((0,), (0,)), ((), ())),
                                 preferred_element_type=jnp.float32)
    cb = c_ref[0] + c_ref[1]
    acc1[...] += lax.dot_general(cb, h2, (((1,), (0,)), ((), ())),
                                 preferred_element_type=jnp.float32)
    accn[...] += jnp.sum(oh, axis=0, keepdims=True)

    @pl.when(i == NP // 256 - 1)
    def _():
        cnt = accn[0]
        icnt = 1.0 / jnp.maximum(cnt, 1.0)
        nz = (cnt > 0).astype(jnp.float32)
        p1 = acc1[...] * icnt[:, None]
        p2 = acc2[...] * icnt[:, None]
        o = jnp.dot(p1, wl_ref[...], preferred_element_type=jnp.float32)
        o += jnp.dot(p2, wr_ref[...], preferred_element_type=jnp.float32)
        o += nz[:, None] * bb_ref[...][None, :] + cb_ref[...][None, :]
        out_ref[...] = o


def _tc23(agg1_p, h1, deg_p, L1, R1, l1b, s1, b1, batchp, C_p,
          WlT, WrT, bb2, clsb):
    return pl.pallas_call(
        _tc23_body,
        grid=(NP // 256,),
        in_specs=[
            pl.BlockSpec((8, 256, LN), lambda i: (0, i, 0)),
            pl.BlockSpec((256, 512), lambda i: (i, 0)),
            pl.BlockSpec((2, 256), lambda i: (0, i)),
            pl.BlockSpec((4, LN, 512), lambda i: (0, 0, 0)),
            pl.BlockSpec((512, 512), lambda i: (0, 0)),
            pl.BlockSpec((512,), lambda i: (0,)),
            pl.BlockSpec((512,), lambda i: (0,)),
            pl.BlockSpec((512,), lambda i: (0,)),
            pl.BlockSpec((256,), lambda i: (i,)),
            pl.BlockSpec((2, G, 256), lambda i: (0, 0, i)),
            pl.BlockSpec((512, 16), lambda i: (0, 0)),
            pl.BlockSpec((512, 16), lambda i: (0, 0)),
            pl.BlockSpec((16,), lambda i: (0,)),
            pl.BlockSpec((16,), lambda i: (0,)),
        ],
        out_specs=pl.BlockSpec((G, 16), lambda i: (0, 0)),
        out_shape=jax.ShapeDtypeStruct((G, 16), jnp.float32),
        scratch_shapes=[
            pltpu.VMEM((G, 512), jnp.float32),
            pltpu.VMEM((G, 512), jnp.float32),
            pltpu.VMEM((1, G), jnp.float32),
        ],
    )(agg1_p, h1, deg_p, L1, R1, l1b, s1, b1, batchp, C_p,
      WlT, WrT, bb2, clsb)


def kernel(x, edge_index, edge_attr, batch, w_emb, b_emb,
           conv0_l_w, conv0_l_b, conv0_r_w,
           conv1_l_w, conv1_l_b, conv1_r_w,
           conv2_l_w, conv2_l_b, conv2_r_w,
           bn0_g, bn0_b, bn1_g, bn1_b, cls_w, cls_b):
    E = edge_index.shape[1]
    src = edge_index[0].astype(jnp.int32)
    dst = edge_index[1].astype(jnp.int32)
    npad = E_PAD - E
    src_p = jnp.concatenate(
        [src, N + (jnp.arange(npad, dtype=jnp.int32) % 16)]
    ).reshape(TILES * NB, LN)
    dst_p = jnp.concatenate(
        [dst, jnp.zeros((npad,), jnp.int32)]).reshape(TILES * NB, LN)
    eones = jnp.concatenate(
        [jnp.ones((E,), jnp.float32), jnp.zeros((npad,), jnp.float32)]
    ).reshape(TILES * NB, LN)
    xp = jnp.pad(x, ((0, NP - N), (0, 0)))
    xp2 = xp.reshape(2 * NP, LN)
    batchp = jnp.pad(batch.astype(jnp.int32), (0, NP - N), constant_values=G)
    z2d = jnp.zeros((64, LN), jnp.float32)
    z1d = jnp.zeros((4096,), jnp.float32)

    inv = 1.0 / np.sqrt(1.0 + EPS)
    L0 = jnp.concatenate([conv0_l_w[:, :10] @ w_emb, conv0_l_w[:, 10:]], axis=1)
    R0 = jnp.concatenate([conv0_r_w[:, :10] @ w_emb, conv0_r_w[:, 10:]], axis=1)
    cbd = conv0_l_w[:, :10] @ b_emb
    cbc = conv0_r_w[:, :10] @ b_emb + conv0_l_b
    s0 = bn0_g * inv
    s1 = bn1_g * inv
    WlT = (cls_w @ conv2_l_w).T
    WrT = (cls_w @ conv2_r_w).T
    bb2 = cls_w @ conv2_l_b

    deg_p, C_p = _sc_hist(src_p, dst_p, eones, batchp, z1d)
    aggx_p = _sc_agg2(xp2, src_p, dst_p, z2d)
    h1 = _tc1(aggx_p.reshape(4, NP, LN), xp, deg_p,
              L0.T.reshape(2, LN, 512), R0.T, cbd, cbc, s0, bn0_b)
    agg1_p = _sc_agg4(h1.reshape(4 * NP, LN), src_p, dst_p, z2d)
    out = _tc23(agg1_p.reshape(8, NP, LN), h1, deg_p,
                conv1_l_w.T.reshape(4, LN, 512), conv1_r_w.T, conv1_l_b,
                s1, bn1_b, batchp, C_p.reshape(2, G, NP),
                WlT, WrT, bb2, cls_b)
    return out


# speedup vs baseline: 1.3625x; 1.0003x over previous
import jax
import jax.numpy as jnp
import numpy as np
from jax import lax
from jax.experimental import pallas as pl
from jax.experimental.pallas import tpu as pltpu
from jax.experimental.pallas import tpu_sc as plsc

N = 10000
G = 64
EPS = 1e-5
NP = 10240
LN = 128
TILES = 32
EPT = 5120
NB = EPT // LN
E_PAD = TILES * EPT
ROWS_PT = NP // 16
CPT = G * NP // 16

_mesh = plsc.VectorSubcoreMesh(core_axis_name="c", subcore_axis_name="s")


def _sc_hist_body(srcr, dstr, eor, batchr, z1dr,
                  deg_o, c_o,
                  src_v, dst_v, eo_v, cidx_v, gk_v, z1d_v,
                  hsem0, hsem1, cS, degS):
    c = lax.axis_index("c")
    s = lax.axis_index("s")
    t = c * 16 + s

    pltpu.sync_copy(srcr.at[pl.ds(t * NB, NB)], src_v)
    pltpu.sync_copy(dstr.at[pl.ds(t * NB, NB)], dst_v)
    pltpu.sync_copy(eor.at[pl.ds(t * NB, NB)], eo_v)
    pltpu.sync_copy(z1dr, z1d_v)

    def gk_fire(j, _):
        pltpu.async_copy(batchr.at[dst_v.at[j]], gk_v.at[j], hsem0)
        return 0

    lax.fori_loop(0, NB, gk_fire, 0)

    def gk_drain(j, _):
        pltpu.make_async_copy(batchr.at[dst_v.at[j]], gk_v.at[j],
                              hsem0).wait()
        return 0

    lax.fori_loop(0, NB, gk_drain, 0)

    def cidx_body(i, _):
        j = i // 8
        k = (i % 8) * 16
        cidx_v[j, pl.ds(k, 16)] = (gk_v[j, pl.ds(k, 16)] * NP
                                   + src_v[j, pl.ds(k, 16)])
        return 0

    lax.fori_loop(0, NB * 8, cidx_body, 0)

    pltpu.sync_copy(z1d_v.at[pl.ds(0, ROWS_PT)],
                    degS.at[pl.ds(s * ROWS_PT, ROWS_PT)])

    def zc(i, _):
        pltpu.sync_copy(z1d_v, cS.at[pl.ds(s * CPT + i * 4096, 4096)])
        return 0

    lax.fori_loop(0, CPT // 4096, zc, 0)
    plsc.subcore_barrier()

    def dc_fire(j, _):
        pltpu.async_copy(eo_v.at[j], degS.at[dst_v.at[j]], hsem0, add=True)
        pltpu.async_copy(eo_v.at[j], cS.at[cidx_v.at[j]], hsem1, add=True)
        return 0

    lax.fori_loop(0, NB, dc_fire, 0)

    def dc_drain(j, _):
        pltpu.make_async_copy(eo_v.at[j], degS.at[dst_v.at[j]],
                              hsem0).wait()
        pltpu.make_async_copy(eo_v.at[j], cS.at[cidx_v.at[j]],
                              hsem1).wait()
        return 0

    lax.fori_loop(0, NB, dc_drain, 0)
    plsc.subcore_barrier()
    pltpu.sync_copy(degS.at[pl.ds(s * ROWS_PT, ROWS_PT)],
                    deg_o.at[c, pl.ds(s * ROWS_PT, ROWS_PT)])

    def wc(i, _):
        off = s * CPT + i * 4096
        pltpu.sync_copy(cS.at[pl.ds(off, 4096)], c_o.at[c, pl.ds(off, 4096)])
        return 0

    lax.fori_loop(0, CPT // 4096, wc, 0)


def _sc_hist(src_p, dst_p, eones, batchp, z1d):
    f = pl.kernel(
        _sc_hist_body,
        out_type=(
            jax.ShapeDtypeStruct((2, NP), jnp.float32),
            jax.ShapeDtypeStruct((2, G * NP), jnp.float32),
        ),
        mesh=_mesh,
        scratch_types=[
            pltpu.VMEM((NB, LN), jnp.int32),
            pltpu.VMEM((NB, LN), jnp.int32),
            pltpu.VMEM((NB, LN), jnp.float32),
            pltpu.VMEM((NB, LN), jnp.int32),
            pltpu.VMEM((NB, LN), jnp.int32),
            pltpu.VMEM((4096,), jnp.float32),
            pltpu.SemaphoreType.DMA,
            pltpu.SemaphoreType.DMA,
            pltpu.VMEM_SHARED((G * NP,), jnp.float32),
            pltpu.VMEM_SHARED((NP,), jnp.float32),
        ],
    )
    return f(src_p, dst_p, eones, batchp, z1d)


def _make_sc_agg(nch, dtype=jnp.float32, W=LN):
    def body(table, srcr, dstr, z2dr, agg_o,
             src_v, dst_v, gidx_v, rows0, rows1, gsem0, gsem1, accS):
        c = lax.axis_index("c")
        s = lax.axis_index("s")
        t = c * 16 + s

        pltpu.sync_copy(srcr.at[pl.ds(t * NB, NB)], src_v)
        pltpu.sync_copy(dstr.at[pl.ds(t * NB, NB)], dst_v)

        for ch in range(nch):
            def gidx_body(i, _):
                j = i // 8
                k = (i % 8) * 16
                gidx_v[j, pl.ds(k, 16)] = src_v[j, pl.ds(k, 16)] * nch + ch
                return 0

            lax.fori_loop(0, NB * 8, gidx_body, 0)

            def zacc(i, _):
                pltpu.sync_copy(z2dr,
                                accS.at[pl.ds(s * ROWS_PT + i * 64, 64), :])
                return 0

            lax.fori_loop(0, ROWS_PT // 64, zacc, 0)
            plsc.subcore_barrier()

            pltpu.async_copy(table.at[gidx_v.at[0]], rows0, gsem0)

            def grp(g, _):
                j0 = 2 * g
                j1 = 2 * g + 1
                pltpu.async_copy(table.at[gidx_v.at[j1]], rows1, gsem1)
                pltpu.make_async_copy(table.at[gidx_v.at[j0]], rows0,
                                      gsem0).wait()
                pltpu.sync_copy(rows0, accS.at[dst_v.at[j0]], add=True)

                @pl.when(j1 + 1 < NB)
                def _():
                    pltpu.async_copy(table.at[gidx_v.at[j1 + 1]], rows0,
                                     gsem0)

                pltpu.make_async_copy(table.at[gidx_v.at[j1]], rows1,
                                      gsem1).wait()
                pltpu.sync_copy(rows1, accS.at[dst_v.at[j1]], add=True)
                return 0

            lax.fori_loop(0, NB // 2, grp, 0)
            plsc.subcore_barrier()
            pltpu.sync_copy(accS.at[pl.ds(s * ROWS_PT, ROWS_PT), :],
                            agg_o.at[c, ch, pl.ds(s * ROWS_PT, ROWS_PT), :])

    def run(table, src_p, dst_p, z2d):
        f = pl.kernel(
            body,
            out_type=jax.ShapeDtypeStruct((2, nch, NP, W), dtype),
            mesh=_mesh,
            scratch_types=[
                pltpu.VMEM((NB, LN), jnp.int32),
                pltpu.VMEM((NB, LN), jnp.int32),
                pltpu.VMEM((NB, LN), jnp.int32),
                pltpu.VMEM((LN, W), dtype),
                pltpu.VMEM((LN, W), dtype),
                pltpu.SemaphoreType.DMA,
                pltpu.SemaphoreType.DMA,
                pltpu.VMEM_SHARED((NP, W), dtype),
            ],
        )
        return f(table, src_p, dst_p, z2d)

    return run


_sc_agg2 = _make_sc_agg(2)
_sc_agg4 = _make_sc_agg(4)


def _tc1_body(agg_ref, xp_ref, deg_ref, L0_ref, R0_ref, cbd_ref, cbc_ref,
              s0_ref, b0_ref, h1_ref):
    i = pl.program_id(0)
    a0 = (agg_ref[0] + agg_ref[2]).astype(jnp.bfloat16)
    a1 = (agg_ref[1] + agg_ref[3]).astype(jnp.bfloat16)
    z = jnp.dot(a0, L0_ref[...][0], preferred_element_type=jnp.float32)
    z += jnp.dot(a1, L0_ref[...][1], preferred_element_type=jnp.float32)
    z += jnp.dot(xp_ref[...].astype(jnp.bfloat16), R0_ref[...],
                 preferred_element_type=jnp.float32)
    deg = deg_ref[0] + deg_ref[1]
    z += deg[:, None] * cbd_ref[...][None, :] + cbc_ref[...][None, :]
    h = jax.nn.relu(z * s0_ref[...][None, :] + b0_ref[...][None, :])
    rid = i * 256 + lax.broadcasted_iota(jnp.int32, (256, 1), 0)
    h1_ref[...] = jnp.where(rid < N, h, 0.0)


def _tc1(aggx_p, xp, deg_p, L0, R0, cbd, cbc, s0, b0):
    return pl.pallas_call(
        _tc1_body,
        grid=(NP // 256,),
        in_specs=[
            pl.BlockSpec((4, 256, LN), lambda i: (0, i, 0)),
            pl.BlockSpec((256, 256), lambda i: (i, 0)),
            pl.BlockSpec((2, 256), lambda i: (0, i)),
            pl.BlockSpec((2, LN, 512), lambda i: (0, 0, 0)),
            pl.BlockSpec((256, 512), lambda i: (0, 0)),
            pl.BlockSpec((512,), lambda i: (0,)),
            pl.BlockSpec((512,), lambda i: (0,)),
            pl.BlockSpec((512,), lambda i: (0,)),
            pl.BlockSpec((512,), lambda i: (0,)),
        ],
        out_specs=pl.BlockSpec((256, 512), lambda i: (i, 0)),
        out_shape=jax.ShapeDtypeStruct((NP, 512), jnp.float32),
    )(aggx_p, xp, deg_p, L0, R0, cbd, cbc, s0, b0)


def _tc23_body(agg_ref, h1_ref, deg_ref, L1_ref, R1_ref, l1b_ref,
               s1_ref, b1_ref, batch_ref, c_ref, wl_ref, wr_ref, bb_ref,
               cb_ref, out_ref, acc1, acc2, accn):
    i = pl.program_id(0)

    @pl.when(i == 0)
    def _():
        acc1[...] = jnp.zeros_like(acc1)
        acc2[...] = jnp.zeros_like(acc2)
        accn[...] = jnp.zeros_like(accn)

    deg = deg_ref[0] + deg_ref[1]
    idg = 1.0 / jnp.maximum(deg, 1.0)
    z = jnp.dot(h1_ref[...].astype(jnp.bfloat16), R1_ref[...],
                preferred_element_type=jnp.float32)
    for ch in range(4):
        a = ((agg_ref[ch] + agg_ref[ch + 4]) * idg[:, None]
             ).astype(jnp.bfloat16)
        z += jnp.dot(a, L1_ref[...][ch], preferred_element_type=jnp.float32)
    z += l1b_ref[...][None, :]
    h = jax.nn.relu(z * s1_ref[...][None, :] + b1_ref[...][None, :])
    rid = i * 256 + lax.broadcasted_iota(jnp.int32, (256, 1), 0)
    h2 = jnp.where(rid < N, h, 0.0)

    bb = batch_ref[...]
    gi = lax.broadcasted_iota(jnp.int32, (256, G), 1)
    oh = (bb[:, None] == gi).astype(jnp.float32)
    acc2[...] += lax.dot_general(oh, h2, (((0,), (0,)), ((), ())),
                                 preferred_element_type=jnp.float32)
    cb = c_ref[0] + c_ref[1]
    acc1[...] += lax.dot_general(cb, h2, (((1,), (0,)), ((), ())),
                                 preferred_element_type=jnp.float32)
    accn[...] += jnp.sum(oh, axis=0, keepdims=True)

    @pl.when(i == NP // 256 - 1)
    def _():
        cnt = accn[0]
        icnt = 1.0 / jnp.maximum(cnt, 1.0)
        nz = (cnt > 0).astype(jnp.float32)
        p1 = acc1[...] * icnt[:, None]
        p2 = acc2[...] * icnt[:, None]
        o = jnp.dot(p1, wl_ref[...], preferred_element_type=jnp.float32)
        o += jnp.dot(p2, wr_ref[...], preferred_element_type=jnp.float32)
        o += nz[:, None] * bb_ref[...][None, :] + cb_ref[...][None, :]
        out_ref[...] = o


def _tc23(agg1_p, h1, deg_p, L1, R1, l1b, s1, b1, batchp, C_p,
          WlT, WrT, bb2, clsb):
    return pl.pallas_call(
        _tc23_body,
        grid=(NP // 256,),
        in_specs=[
            pl.BlockSpec((8, 256, LN), lambda i: (0, i, 0)),
            pl.BlockSpec((256, 512), lambda i: (i, 0)),
            pl.BlockSpec((2, 256), lambda i: (0, i)),
            pl.BlockSpec((4, LN, 512), lambda i: (0, 0, 0)),
            pl.BlockSpec((512, 512), lambda i: (0, 0)),
            pl.BlockSpec((512,), lambda i: (0,)),
            pl.BlockSpec((512,), lambda i: (0,)),
            pl.BlockSpec((512,), lambda i: (0,)),
            pl.BlockSpec((256,), lambda i: (i,)),
            pl.BlockSpec((2, G, 256), lambda i: (0, 0, i)),
            pl.BlockSpec((512, 16), lambda i: (0, 0)),
            pl.BlockSpec((512, 16), lambda i: (0, 0)),
            pl.BlockSpec((16,), lambda i: (0,)),
            pl.BlockSpec((16,), lambda i: (0,)),
        ],
        out_specs=pl.BlockSpec((G, 16), lambda i: (0, 0)),
        out_shape=jax.ShapeDtypeStruct((G, 16), jnp.float32),
        scratch_shapes=[
            pltpu.VMEM((G, 512), jnp.float32),
            pltpu.VMEM((G, 512), jnp.float32),
            pltpu.VMEM((1, G), jnp.float32),
        ],
    )(agg1_p, h1, deg_p, L1, R1, l1b, s1, b1, batchp, C_p,
      WlT, WrT, bb2, clsb)


def kernel(x, edge_index, edge_attr, batch, w_emb, b_emb,
           conv0_l_w, conv0_l_b, conv0_r_w,
           conv1_l_w, conv1_l_b, conv1_r_w,
           conv2_l_w, conv2_l_b, conv2_r_w,
           bn0_g, bn0_b, bn1_g, bn1_b, cls_w, cls_b):
    E = edge_index.shape[1]
    src = edge_index[0].astype(jnp.int32)
    dst = edge_index[1].astype(jnp.int32)
    npad = E_PAD - E
    src_p = jnp.concatenate(
        [src, N + (jnp.arange(npad, dtype=jnp.int32) % 16)]
    ).reshape(TILES * NB, LN)
    dst_p = jnp.concatenate(
        [dst, jnp.zeros((npad,), jnp.int32)]).reshape(TILES * NB, LN)
    eones = jnp.concatenate(
        [jnp.ones((E,), jnp.float32), jnp.zeros((npad,), jnp.float32)]
    ).reshape(TILES * NB, LN)
    xp = jnp.pad(x, ((0, NP - N), (0, 0)))
    xp2 = xp.reshape(2 * NP, LN)
    batchp = jnp.pad(batch.astype(jnp.int32), (0, NP - N), constant_values=G)
    z2d = jnp.zeros((64, LN), jnp.float32)
    z1d = jnp.zeros((4096,), jnp.float32)

    inv = 1.0 / np.sqrt(1.0 + EPS)
    L0 = jnp.concatenate([conv0_l_w[:, :10] @ w_emb, conv0_l_w[:, 10:]], axis=1)
    R0 = jnp.concatenate([conv0_r_w[:, :10] @ w_emb, conv0_r_w[:, 10:]], axis=1)
    cbd = conv0_l_w[:, :10] @ b_emb
    cbc = conv0_r_w[:, :10] @ b_emb + conv0_l_b
    s0 = bn0_g * inv
    s1 = bn1_g * inv
    WlT = (cls_w @ conv2_l_w).T
    WrT = (cls_w @ conv2_r_w).T
    bb2 = cls_w @ conv2_l_b

    deg_p, C_p = _sc_hist(src_p, dst_p, eones, batchp, z1d)
    aggx_p = _sc_agg2(xp2, src_p, dst_p, z2d)
    h1 = _tc1(aggx_p.reshape(4, NP, LN), xp, deg_p,
              L0.T.reshape(2, LN, 512).astype(jnp.bfloat16),
              R0.T.astype(jnp.bfloat16), cbd, cbc, s0, bn0_b)
    agg1_p = _sc_agg4(h1.reshape(4 * NP, LN), src_p, dst_p, z2d)
    out = _tc23(agg1_p.reshape(8, NP, LN), h1, deg_p,
                conv1_l_w.T.reshape(4, LN, 512).astype(jnp.bfloat16),
                conv1_r_w.T.astype(jnp.bfloat16), conv1_l_b,
                s1, bn1_b, batchp, C_p.reshape(2, G, NP),
                WlT, WrT, bb2, cls_b)
    return out
